# Initial kernel scaffold; baseline (speedup 1.0000x reference)
import jax
import jax.numpy as jnp
from jax.experimental import pallas as pl


def kernel(h, edge_index, edge_relation, node_momentum_signature, node_role, node_mass_features, edge_channel, rel_emb, role_emb, channel_emb, eW1, eb1, eW2, eb2, nW1, nb1, nW2, nb2, ln_g, ln_b):
    raise NotImplementedError("write your pallas kernel here")


# speedup vs baseline: 2.1209x; 2.1209x over previous
import functools

import jax
import jax.numpy as jnp
from jax.experimental import pallas as pl
from jax.experimental.pallas import tpu as pltpu

H = 128
_EB = 2560
_NB = 2000


def _slog(x):
    return jnp.sign(x) * jnp.log1p(jnp.abs(x))


def _silu(x):
    return x * jax.nn.sigmoid(x)


def _edge_body(x0_ref, sig_ref, w12_ref, ew2_ref, eb2_ref, out_ref):
    s = sig_ref[:, 0:4]
    d = sig_ref[:, 4:8]
    sm = s + d
    df = d - s
    stats = jnp.concatenate([
        jnp.sum(s * d, -1, keepdims=True),
        jnp.sum(df * df, -1, keepdims=True),
        jnp.sum(s * s, -1, keepdims=True),
        jnp.sum(d * d, -1, keepdims=True)], -1)
    feats = jnp.concatenate([_slog(sm), _slog(jnp.abs(df)), _slog(stats)], -1)
    x = x0_ref[...] + jnp.dot(feats, w12_ref[...],
                              preferred_element_type=jnp.float32)
    a = _silu(x)
    m = _silu(jnp.dot(a, ew2_ref[...], preferred_element_type=jnp.float32)
              + eb2_ref[...])
    out_ref[...] = m


def _edge_mlp(x0, sigs, w12, ew2, eb2):
    E = x0.shape[0]
    grid = E // _EB
    return pl.pallas_call(
        _edge_body,
        grid=(grid,),
        in_specs=[
            pl.BlockSpec((_EB, H), lambda i: (i, 0)),
            pl.BlockSpec((_EB, 8), lambda i: (i, 0)),
            pl.BlockSpec((12, H), lambda i: (0, 0)),
            pl.BlockSpec((H, H), lambda i: (0, 0)),
            pl.BlockSpec((1, H), lambda i: (0, 0)),
        ],
        out_specs=pl.BlockSpec((_EB, H), lambda i: (i, 0)),
        out_shape=jax.ShapeDtypeStruct((E, H), jnp.float32),
    )(x0, sigs, w12, ew2, eb2.reshape(1, H))


def _node_body(h_ref, agg_ref, pre_ref, w1h_ref, w1a_ref, w2_ref, nb2_ref,
               g_ref, b_ref, out_ref):
    x = _silu(jnp.dot(h_ref[...], w1h_ref[...],
                      preferred_element_type=jnp.float32)
              + jnp.dot(agg_ref[...], w1a_ref[...],
                        preferred_element_type=jnp.float32)
              + pre_ref[...])
    y = h_ref[...] + jnp.dot(x, w2_ref[...],
                             preferred_element_type=jnp.float32) + nb2_ref[...]
    mu = jnp.mean(y, -1, keepdims=True)
    yc = y - mu
    var = jnp.mean(yc * yc, -1, keepdims=True)
    out_ref[...] = yc * jax.lax.rsqrt(var + 1e-5) * g_ref[...] + b_ref[...]


def _node_mlp(h, agg, pre, w1h, w1a, w2, nb2, g, b):
    N = h.shape[0]
    grid = N // _NB
    full = lambda i: (0, 0)
    row = lambda i: (i, 0)
    return pl.pallas_call(
        _node_body,
        grid=(grid,),
        in_specs=[
            pl.BlockSpec((_NB, H), row),
            pl.BlockSpec((_NB, H), row),
            pl.BlockSpec((_NB, H), row),
            pl.BlockSpec((H, H), full),
            pl.BlockSpec((H, H), full),
            pl.BlockSpec((H, H), full),
            pl.BlockSpec((1, H), full),
            pl.BlockSpec((1, H), full),
            pl.BlockSpec((1, H), full),
        ],
        out_specs=pl.BlockSpec((_NB, H), row),
        out_shape=jax.ShapeDtypeStruct((N, H), jnp.float32),
    )(h, agg, pre, w1h, w1a, w2, nb2.reshape(1, H), g.reshape(1, H),
      b.reshape(1, H))


def kernel(h, edge_index, edge_relation, node_momentum_signature, node_role,
           node_mass_features, edge_channel, rel_emb, role_emb, channel_emb,
           eW1, eb1, eW2, eb2, nW1, nb1, nW2, nb2, ln_g, ln_b):
    N = h.shape[0]
    sig = node_momentum_signature
    mass = node_mass_features
    src, dst = edge_index[0], edge_index[1]
    ssig_l = _slog(sig)

    Asrc = (h @ eW1[0:128] + ssig_l @ eW1[256:260] + mass @ eW1[276:278]
            + (role_emb @ eW1[296:304])[node_role])
    Adst = (h @ eW1[128:256] + ssig_l @ eW1[260:264] + mass @ eW1[278:280]
            + (role_emb @ eW1[304:312])[node_role])
    RelP = rel_emb @ eW1[280:296] + eb1
    ChP = channel_emb @ eW1[312:320]
    W12 = eW1[264:276]

    X0 = Asrc[src] + Adst[dst] + RelP[edge_relation] + ChP[edge_channel]
    sigs = jnp.concatenate([sig[src], sig[dst]], axis=-1)

    m = _edge_mlp(X0, sigs, W12, eW2, eb2)
    agg = jax.ops.segment_sum(m, dst, num_segments=N)

    pre = (ssig_l @ nW1[256:260] + mass @ nW1[260:262]
           + (role_emb @ nW1[262:270])[node_role] + nb1)
    return _node_mlp(h, agg, pre, nW1[0:128], nW1[128:256], nW2, nb2,
                     ln_g, ln_b)


# speedup vs baseline: 9.1846x; 4.3306x over previous
import functools

import jax
import jax.numpy as jnp
from jax import lax
from jax.experimental import pallas as pl
from jax.experimental.pallas import tpu as pltpu
from jax.experimental.pallas import tpu_sc as plsc

H = 128
_EB = 2560
_NB = 2000
_NC = 2
_NS = 16
_NW = _NC * _NS
_GC = 400
_SC_CHUNK = 200
_NP = 10240


def _slog(x):
    return jnp.sign(x) * jnp.log1p(jnp.abs(x))


def _silu(x):
    return x * jax.nn.sigmoid(x)


def _pre_body(h_ref, smr_ref, w1s_ref, w1d_ref, ws_ref, wd_ref, wp_ref,
              asrc_ref, adst_ref, pre_ref):
    h = h_ref[...]
    smr = smr_ref[...]
    sl = _slog(smr[:, 0:4])
    f = jnp.concatenate([sl, smr[:, 4:16]], axis=-1)
    asrc_ref[...] = (jnp.dot(h, w1s_ref[...], preferred_element_type=jnp.float32)
                     + jnp.dot(f, ws_ref[...], preferred_element_type=jnp.float32))
    adst_ref[...] = (jnp.dot(h, w1d_ref[...], preferred_element_type=jnp.float32)
                     + jnp.dot(f, wd_ref[...], preferred_element_type=jnp.float32))
    pre_ref[...] = jnp.dot(f, wp_ref[...], preferred_element_type=jnp.float32)


def _pre_kernel(h, smr, w1s, w1d, ws, wd, wp):
    N = h.shape[0]
    grid = N // _NB
    row = lambda i: (i, 0)
    full = lambda i: (0, 0)
    out = jax.ShapeDtypeStruct((N, H), jnp.float32)
    return pl.pallas_call(
        _pre_body,
        grid=(grid,),
        in_specs=[
            pl.BlockSpec((_NB, H), row),
            pl.BlockSpec((_NB, 16), row),
            pl.BlockSpec((H, H), full),
            pl.BlockSpec((H, H), full),
            pl.BlockSpec((16, H), full),
            pl.BlockSpec((16, H), full),
            pl.BlockSpec((16, H), full),
        ],
        out_specs=[pl.BlockSpec((_NB, H), row)] * 3,
        out_shape=[out, out, out],
    )(h, smr, w1s, w1d, ws, wd, wp)


def _gather_rows(src, dst, ta, tb):
    E = src.shape[0]
    ew = E // _NW
    mesh = plsc.VectorSubcoreMesh(core_axis_name="c", subcore_axis_name="s")

    @functools.partial(
        pl.kernel,
        out_type=[
            jax.ShapeDtypeStruct((E, H), jnp.float32),
            jax.ShapeDtypeStruct((E, H), jnp.float32),
        ],
        mesh=mesh,
        scratch_types=[
            pltpu.VMEM((_GC,), jnp.int32),
            pltpu.VMEM((_GC,), jnp.int32),
            pltpu.VMEM((_GC, H), jnp.float32),
            pltpu.VMEM((_GC, H), jnp.float32),
            pltpu.SemaphoreType.DMA,
            pltpu.SemaphoreType.DMA,
        ],
    )
    def gk(src_h, dst_h, ta_h, tb_h, ga_h, gb_h,
           idxs_v, idxd_v, rows_v, rows2_v, sem, sem2):
        wid = lax.axis_index("s") * _NC + lax.axis_index("c")
        base = wid * ew

        def chunk(i, _):
            off = base + i * _GC
            pltpu.sync_copy(src_h.at[pl.ds(off, _GC)], idxs_v)
            pltpu.sync_copy(dst_h.at[pl.ds(off, _GC)], idxd_v)
            cpa = pltpu.async_copy(ta_h.at[idxs_v], rows_v, sem)
            cpb = pltpu.async_copy(tb_h.at[idxd_v], rows2_v, sem2)
            cpa.wait()
            pltpu.sync_copy(rows_v, ga_h.at[pl.ds(off, _GC)])
            cpb.wait()
            pltpu.sync_copy(rows2_v, gb_h.at[pl.ds(off, _GC)])
            return 0

        lax.fori_loop(0, ew // _GC, chunk, 0)

    return gk(src, dst, ta, tb)


def _gather_sigs(src, dst, sigp):
    E = src.shape[0]
    ew = E // _NW
    mesh = plsc.VectorSubcoreMesh(core_axis_name="c", subcore_axis_name="s")

    @functools.partial(
        pl.kernel,
        out_type=[
            jax.ShapeDtypeStruct((E, 16), jnp.float32),
            jax.ShapeDtypeStruct((E, 16), jnp.float32),
        ],
        mesh=mesh,
        scratch_types=[
            pltpu.VMEM((_GC,), jnp.int32),
            pltpu.VMEM((_GC,), jnp.int32),
            pltpu.VMEM((_GC, 16), jnp.float32),
            pltpu.VMEM((_GC, 16), jnp.float32),
            pltpu.SemaphoreType.DMA,
            pltpu.SemaphoreType.DMA,
        ],
        compiler_params=pltpu.CompilerParams(use_tc_tiling_on_sc=False),
    )
    def gk(src_h, dst_h, sg_h, ss_h, sd_h,
           idxs_v, idxd_v, srows_v, srows2_v, sem, sem2):
        wid = lax.axis_index("s") * _NC + lax.axis_index("c")
        base = wid * ew

        def chunk(i, _):
            off = base + i * _GC
            pltpu.sync_copy(src_h.at[pl.ds(off, _GC)], idxs_v)
            pltpu.sync_copy(dst_h.at[pl.ds(off, _GC)], idxd_v)
            cpa = pltpu.async_copy(sg_h.at[idxs_v], srows_v, sem)
            cpb = pltpu.async_copy(sg_h.at[idxd_v], srows2_v, sem2)
            cpa.wait()
            pltpu.sync_copy(srows_v, ss_h.at[pl.ds(off, _GC)])
            cpb.wait()
            pltpu.sync_copy(srows2_v, sd_h.at[pl.ds(off, _GC)])
            return 0

        lax.fori_loop(0, ew // _GC, chunk, 0)

    return gk(src, dst, sigp)


def _scatter_sc(m, dst, zrows):
    E = m.shape[0]
    half = E // _NC
    ew = half // _NS
    rows_per_tile = _NP // _NS
    mesh = plsc.VectorSubcoreMesh(core_axis_name="c", subcore_axis_name="s")

    @functools.partial(
        pl.kernel,
        out_type=jax.ShapeDtypeStruct((_NC, _NP, H), jnp.float32),
        mesh=mesh,
        scratch_types=[
            pltpu.VMEM((_SC_CHUNK,), jnp.int32),
            pltpu.VMEM((_SC_CHUNK, H), jnp.float32),
            pltpu.VMEM_SHARED((_NP, H), jnp.float32),
        ],
    )
    def sk(m_h, dst_h, z_h, out_h, idx_v, rows_v, agg_s):
        core = lax.axis_index("c")
        sid = lax.axis_index("s")
        rbase = sid * rows_per_tile
        pltpu.sync_copy(z_h.at[pl.ds(rbase, rows_per_tile)],
                        agg_s.at[pl.ds(rbase, rows_per_tile)])
        plsc.subcore_barrier()
        base = core * half + sid * ew

        def chunk(i, _):
            off = base + i * _SC_CHUNK
            pltpu.sync_copy(m_h.at[pl.ds(off, _SC_CHUNK)], rows_v)
            pltpu.sync_copy(dst_h.at[pl.ds(off, _SC_CHUNK)], idx_v)
            pltpu.sync_copy(rows_v, agg_s.at[idx_v], add=True)
            return 0

        lax.fori_loop(0, ew // _SC_CHUNK, chunk, 0)
        plsc.subcore_barrier()
        pltpu.sync_copy(agg_s.at[pl.ds(rbase, rows_per_tile)],
                        out_h.at[core, pl.ds(rbase, rows_per_tile)])

    return sk(m, dst, zrows)


def _edge_body(ga_ref, gb_ref, ss_ref, sd_ref, rf_ref, cf_ref, w24_ref,
               ew2_ref, eb2_ref, out_ref):
    s = ss_ref[:, 0:4]
    d = sd_ref[:, 0:4]
    sm = s + d
    df = d - s
    stats = jnp.concatenate([
        jnp.sum(s * d, -1, keepdims=True),
        jnp.sum(df * df, -1, keepdims=True),
        jnp.sum(s * s, -1, keepdims=True),
        jnp.sum(d * d, -1, keepdims=True)], -1)
    B = s.shape[0]
    ohr = (rf_ref[...] == lax.broadcasted_iota(jnp.int32, (B, 8), 1)
           .astype(jnp.float32)).astype(jnp.float32)
    ohc = (cf_ref[...] == lax.broadcasted_iota(jnp.int32, (B, 4), 1)
           .astype(jnp.float32)).astype(jnp.float32)
    feats = jnp.concatenate(
        [_slog(sm), _slog(jnp.abs(df)), _slog(stats), ohr, ohc], -1)
    x = (ga_ref[...] + gb_ref[...]
         + jnp.dot(feats, w24_ref[...], preferred_element_type=jnp.float32))
    a = _silu(x)
    out_ref[...] = _silu(
        jnp.dot(a, ew2_ref[...], preferred_element_type=jnp.float32)
        + eb2_ref[...])


def _edge_mlp(ga, gb, ss, sd, relf, chf, w24, ew2, eb2):
    E = ga.shape[0]
    grid = E // _EB
    row = lambda i: (i, 0)
    full = lambda i: (0, 0)
    return pl.pallas_call(
        _edge_body,
        grid=(grid,),
        in_specs=[
            pl.BlockSpec((_EB, H), row),
            pl.BlockSpec((_EB, H), row),
            pl.BlockSpec((_EB, 16), row),
            pl.BlockSpec((_EB, 16), row),
            pl.BlockSpec((_EB, 1), row),
            pl.BlockSpec((_EB, 1), row),
            pl.BlockSpec((24, H), full),
            pl.BlockSpec((H, H), full),
            pl.BlockSpec((1, H), full),
        ],
        out_specs=pl.BlockSpec((_EB, H), row),
        out_shape=jax.ShapeDtypeStruct((E, H), jnp.float32),
    )(ga, gb, ss, sd, relf, chf, w24, ew2, eb2.reshape(1, H))


def _node_body(h_ref, p0_ref, p1_ref, pre_ref, w1h_ref, w1a_ref, w2_ref,
               nb2_ref, g_ref, b_ref, out_ref):
    agg = p0_ref[...] + p1_ref[...]
    x = _silu(jnp.dot(h_ref[...], w1h_ref[...],
                      preferred_element_type=jnp.float32)
              + jnp.dot(agg, w1a_ref[...], preferred_element_type=jnp.float32)
              + pre_ref[...])
    y = h_ref[...] + jnp.dot(x, w2_ref[...],
                             preferred_element_type=jnp.float32) + nb2_ref[...]
    mu = jnp.mean(y, -1, keepdims=True)
    yc = y - mu
    var = jnp.mean(yc * yc, -1, keepdims=True)
    out_ref[...] = yc * jax.lax.rsqrt(var + 1e-5) * g_ref[...] + b_ref[...]


def _node_mlp(h, p0, p1, pre, w1h, w1a, w2, nb2, g, b):
    N = h.shape[0]
    grid = N // _NB
    full = lambda i: (0, 0)
    row = lambda i: (i, 0)
    return pl.pallas_call(
        _node_body,
        grid=(grid,),
        in_specs=[
            pl.BlockSpec((_NB, H), row),
            pl.BlockSpec((_NB, H), row),
            pl.BlockSpec((_NB, H), row),
            pl.BlockSpec((_NB, H), row),
            pl.BlockSpec((H, H), full),
            pl.BlockSpec((H, H), full),
            pl.BlockSpec((H, H), full),
            pl.BlockSpec((1, H), full),
            pl.BlockSpec((1, H), full),
            pl.BlockSpec((1, H), full),
        ],
        out_specs=pl.BlockSpec((_NB, H), row),
        out_shape=jax.ShapeDtypeStruct((N, H), jnp.float32),
    )(h, p0, p1, pre, w1h, w1a, w2, nb2.reshape(1, H), g.reshape(1, H),
      b.reshape(1, H))


def kernel(h, edge_index, edge_relation, node_momentum_signature, node_role,
           node_mass_features, edge_channel, rel_emb, role_emb, channel_emb,
           eW1, eb1, eW2, eb2, nW1, nb1, nW2, nb2, ln_g, ln_b):
    N = h.shape[0]
    sig = node_momentum_signature
    mass = node_mass_features
    src, dst = edge_index[0], edge_index[1]

    roh = jax.nn.one_hot(node_role, 6, dtype=jnp.float32)
    smr = jnp.concatenate(
        [sig, mass, roh, jnp.ones((N, 1), jnp.float32),
         jnp.zeros((N, 3), jnp.float32)], axis=-1)
    zpad = jnp.zeros((4, H), jnp.float32)
    ws = jnp.concatenate([eW1[256:260], eW1[276:278],
                          role_emb @ eW1[296:304], zpad], axis=0)
    wd = jnp.concatenate([eW1[260:264], eW1[278:280],
                          role_emb @ eW1[304:312], zpad], axis=0)
    wp = jnp.concatenate([nW1[256:260], nW1[260:262],
                          role_emb @ nW1[262:270], nb1.reshape(1, H),
                          jnp.zeros((3, H), jnp.float32)], axis=0)
    RelP = rel_emb @ eW1[280:296] + eb1
    ChP = channel_emb @ eW1[312:320]
    w24 = jnp.concatenate([eW1[264:276], RelP, ChP], axis=0)
    sigp = jnp.concatenate([sig, jnp.zeros((N, 12), jnp.float32)], axis=-1)
    relf = edge_relation.astype(jnp.float32).reshape(-1, 1)
    chf = edge_channel.astype(jnp.float32).reshape(-1, 1)

    asrc, adst, pre = _pre_kernel(h, smr, eW1[0:128], eW1[128:256], ws, wd, wp)

    ga, gb = _gather_rows(src, dst, asrc, adst)
    ss, sd = _gather_sigs(src, dst, sigp)

    m = _edge_mlp(ga, gb, ss, sd, relf, chf, w24, eW2, eb2)

    zrows = jnp.zeros((_NP, H), jnp.float32)
    parts = _scatter_sc(m, dst, zrows)

    return _node_mlp(h, parts[0, :N], parts[1, :N], pre, nW1[0:128],
                     nW1[128:256], nW2, nb2, ln_g, ln_b)
